# baseline (speedup 1.0000x reference)
import jax
import jax.numpy as jnp
from jax import lax
from jax.experimental import pallas as pl
from jax.experimental.pallas import tpu as pltpu
from jax.experimental.pallas import tpu_sc as plsc

N_OBJ = 10000
N_ATTR = 10000
D = 128
D_AUG = 144
N_PAD = 10240
E_TOTAL = 320000
NC, NS = 2, 16
EPT = E_TOTAL // (NC * NS)
CHUNK = 128
NFULL = EPT // CHUNK
REM = EPT - NFULL * CHUNK


def _scale_rows(rows_ref, w_ref, n_edges):
    for eg in range(n_edges // 16):
        w_vec = w_ref[pl.ds(eg * 16, 16)]
        for k in range(16):
            wk = w_vec[k]
            row = eg * 16 + k
            for j in range(D_AUG // 16):
                sl = pl.ds(j * 16, 16)
                rows_ref[row, sl] = rows_ref[row, sl] * wk


def _sc_body(attr_hbm, eidx_hbm, w_hbm, zeros_hbm, out_f, out_w,
             acc_sh, aidx, oidx, wbuf, rows, rem_oidx, gsem, ssem, pfsem,
             rsem):
    cid = lax.axis_index("c")
    sid = lax.axis_index("s")
    wid = cid * NS + sid

    rps = N_PAD // NS
    pltpu.sync_copy(zeros_hbm.at[pl.ds(sid * rps, rps)],
                    acc_sh.at[pl.ds(sid * rps, rps)])
    plsc.subcore_barrier()

    base = wid * EPT

    def start_prefetch(g, s):
        off = base + g * CHUNK
        pltpu.async_copy(eidx_hbm.at[1, pl.ds(off, CHUNK)], aidx.at[s],
                         pfsem.at[s])
        pltpu.async_copy(eidx_hbm.at[0, pl.ds(off, CHUNK)], oidx.at[s],
                         pfsem.at[s])
        pltpu.async_copy(w_hbm.at[pl.ds(off, CHUNK)], wbuf.at[s],
                         pfsem.at[s])

    def wait_prefetch(s):
        pltpu.make_async_copy(eidx_hbm.at[1, pl.ds(0, CHUNK)], aidx.at[s],
                              pfsem.at[s]).wait()
        pltpu.make_async_copy(eidx_hbm.at[0, pl.ds(0, CHUNK)], oidx.at[s],
                              pfsem.at[s]).wait()
        pltpu.make_async_copy(w_hbm.at[pl.ds(0, CHUNK)], wbuf.at[s],
                              pfsem.at[s]).wait()

    def start_gather(s, b):
        pltpu.async_copy(attr_hbm.at[0].at[aidx.at[s]], rows.at[b],
                         gsem.at[b])

    def wait_gather(b):
        pltpu.make_async_copy(attr_hbm.at[0, pl.ds(0, CHUNK)], rows.at[b],
                              gsem.at[b]).wait()

    def start_scatter(s, b):
        pltpu.async_copy(rows.at[b], acc_sh.at[oidx.at[s]], ssem.at[b],
                         add=True)

    def wait_scatter(s, b):
        pltpu.make_async_copy(rows.at[b], acc_sh.at[oidx.at[s]],
                              ssem.at[b]).wait()

    start_prefetch(0, 0)
    start_prefetch(1, 1)
    wait_prefetch(0)
    start_gather(0, 0)

    def chunk_body(g, carry):
        b = lax.rem(g, 2)
        nb = 1 - b
        s_cur = lax.rem(g, 3)
        s_nxt = lax.rem(g + 1, 3)
        s_pf = lax.rem(g + 2, 3)

        @pl.when(g + 1 < NFULL)
        def _():
            wait_prefetch(s_nxt)

            @pl.when(g >= 1)
            def _():
                wait_scatter(s_pf, nb)

            start_gather(s_nxt, nb)

        @pl.when(g + 2 < NFULL)
        def _():
            start_prefetch(g + 2, s_pf)

        wait_gather(b)
        _scale_rows(rows.at[b], wbuf.at[s_cur], CHUNK)
        start_scatter(s_cur, b)
        return carry

    lax.fori_loop(0, NFULL, chunk_body, 0)

    wait_scatter((NFULL - 2) % 3, (NFULL - 2) % 2)
    wait_scatter((NFULL - 1) % 3, (NFULL - 1) % 2)

    off = base + NFULL * CHUNK
    pltpu.sync_copy(eidx_hbm.at[1, pl.ds(off, REM)], aidx.at[0, pl.ds(0, REM)])
    pltpu.sync_copy(eidx_hbm.at[0, pl.ds(off, REM)], rem_oidx)
    pltpu.sync_copy(w_hbm.at[pl.ds(off, REM)], wbuf.at[0, pl.ds(0, REM)])
    pltpu.async_copy(attr_hbm.at[0].at[aidx.at[0, pl.ds(0, REM)]],
                     rows.at[0, pl.ds(0, REM)], rsem).wait()
    _scale_rows(rows.at[0], wbuf.at[0], REM)
    pltpu.sync_copy(rows.at[0, pl.ds(0, REM)], acc_sh.at[rem_oidx], add=True)

    plsc.subcore_barrier()
    pltpu.sync_copy(acc_sh.at[pl.ds(sid * rps, rps), pl.ds(0, D)],
                    out_f.at[cid, pl.ds(sid * rps, rps)])
    pltpu.sync_copy(acc_sh.at[pl.ds(sid * rps, rps), pl.ds(D, D_AUG - D)],
                    out_w.at[cid, pl.ds(sid * rps, rps)])


def _sc_aggregate(attr_aug, edge_index, edge_weight, zeros):
    mesh = plsc.VectorSubcoreMesh(core_axis_name="c", subcore_axis_name="s")
    return pl.kernel(
        _sc_body,
        out_type=(jax.ShapeDtypeStruct((NC, N_PAD, D), jnp.float32),
                  jax.ShapeDtypeStruct((NC, N_PAD, D_AUG - D), jnp.float32)),
        mesh=mesh,
        compiler_params=pltpu.CompilerParams(use_tc_tiling_on_sc=False),
        scratch_types=[
            pltpu.VMEM_SHARED((N_PAD, D_AUG), jnp.float32),
            pltpu.VMEM((3, CHUNK), jnp.int32),
            pltpu.VMEM((3, CHUNK), jnp.int32),
            pltpu.VMEM((3, CHUNK), jnp.float32),
            pltpu.VMEM((2, CHUNK, D_AUG), jnp.float32),
            pltpu.VMEM((REM,), jnp.int32),
            pltpu.SemaphoreType.DMA((2,)),
            pltpu.SemaphoreType.DMA((2,)),
            pltpu.SemaphoreType.DMA((3,)),
            pltpu.SemaphoreType.DMA,
        ],
    )(attr_aug, edge_index, edge_weight, zeros)


def _tc_body(accf_ref, accw_ref, obj_ref, wattr_t_ref, battr_ref,
             wproj_t_ref, bproj_ref, wupd_obj_t_ref, wupd_proj_t_ref,
             bupd_ref, out_ref):
    agg_raw = accf_ref[0] + accf_ref[1]
    wsum = accw_ref[0, :, :1] + accw_ref[1, :, :1]
    agg = agg_raw @ wattr_t_ref[...] + wsum * battr_ref[...]
    agg = agg / jnp.maximum(wsum, 1e-6)
    proj = agg @ wproj_t_ref[...] + bproj_ref[...]
    upd = obj_ref[...] @ wupd_obj_t_ref[...] + proj @ wupd_proj_t_ref[...]
    out_ref[0] = jnp.maximum(upd + bupd_ref[...], 0.0)


def _tc_epilogue(accf, accw, flat_obj, W_attr, b_attr, W_proj, b_proj,
                 W_upd, b_upd):
    blk = 2000
    grid = (N_OBJ // blk,)
    return pl.pallas_call(
        _tc_body,
        grid=grid,
        in_specs=[
            pl.BlockSpec((NC, blk, D), lambda i: (0, i, 0)),
            pl.BlockSpec((NC, blk, D_AUG - D), lambda i: (0, i, 0)),
            pl.BlockSpec((blk, D), lambda i: (i, 0)),
            pl.BlockSpec((D, D), lambda i: (0, 0)),
            pl.BlockSpec((1, D), lambda i: (0, 0)),
            pl.BlockSpec((D, D), lambda i: (0, 0)),
            pl.BlockSpec((1, D), lambda i: (0, 0)),
            pl.BlockSpec((D, D), lambda i: (0, 0)),
            pl.BlockSpec((D, D), lambda i: (0, 0)),
            pl.BlockSpec((1, D), lambda i: (0, 0)),
        ],
        out_specs=pl.BlockSpec((1, blk, D), lambda i: (0, i, 0)),
        out_shape=jax.ShapeDtypeStruct((1, N_OBJ, D), jnp.float32),
    )(accf, accw, flat_obj, W_attr.T, b_attr.reshape(1, D), W_proj.T,
      b_proj.reshape(1, D), W_upd[:, :D].T, W_upd[:, D:].T,
      b_upd.reshape(1, D))


@jax.jit
def kernel(object_feats, attr_feats, edge_index, edge_weight,
           W_attr, b_attr, W_proj, b_proj, W_upd, b_upd):
    flat_obj = object_feats.reshape(N_OBJ, D)
    aug = jnp.pad(attr_feats, ((0, 0), (0, 0), (0, D_AUG - D)),
                  constant_values=1.0)
    zeros = jnp.zeros((N_PAD, D_AUG), jnp.float32)

    accf, accw = _sc_aggregate(aug, edge_index, edge_weight, zeros)
    return _tc_epilogue(accf, accw, flat_obj, W_attr, b_attr, W_proj,
                        b_proj, W_upd, b_upd)


# speedup vs baseline: 1.0006x; 1.0006x over previous
import jax
import jax.numpy as jnp
from jax import lax
from jax.experimental import pallas as pl
from jax.experimental.pallas import tpu as pltpu
from jax.experimental.pallas import tpu_sc as plsc

N_OBJ = 10000
N_ATTR = 10000
D = 128
D_AUG = 144
N_PAD = 10240
E_TOTAL = 320000
NC, NS = 2, 16
EPT = E_TOTAL // (NC * NS)
CHUNK = 128
NFULL = EPT // CHUNK
REM = EPT - NFULL * CHUNK


def _scale_rows(rows_ref, w_ref, n_edges):
    for eg in range(n_edges // 16):
        w_vec = w_ref[pl.ds(eg * 16, 16)]
        for k in range(16):
            wk = w_vec[k]
            row = eg * 16 + k
            for j in range(D_AUG // 16):
                sl = pl.ds(j * 16, 16)
                rows_ref[row, sl] = rows_ref[row, sl] * wk


def _sc_body(attr_hbm, eidx_hbm, w_hbm, zeros_hbm, out_f, out_w,
             acc_sh, aidx, oidx, wbuf, rows, rem_oidx, gsem, ssem, pfsem,
             rsem):
    cid = lax.axis_index("c")
    sid = lax.axis_index("s")
    wid = cid * NS + sid

    rps = N_PAD // NS
    pltpu.sync_copy(zeros_hbm.at[pl.ds(sid * rps, rps)],
                    acc_sh.at[pl.ds(sid * rps, rps)])
    plsc.subcore_barrier()

    base = wid * EPT

    def start_prefetch(g, s):
        off = base + g * CHUNK
        pltpu.async_copy(eidx_hbm.at[1, pl.ds(off, CHUNK)], aidx.at[s],
                         pfsem.at[s])
        pltpu.async_copy(eidx_hbm.at[0, pl.ds(off, CHUNK)], oidx.at[s],
                         pfsem.at[s])
        pltpu.async_copy(w_hbm.at[pl.ds(off, CHUNK)], wbuf.at[s],
                         pfsem.at[s])

    def wait_prefetch(s):
        pltpu.make_async_copy(eidx_hbm.at[1, pl.ds(0, CHUNK)], aidx.at[s],
                              pfsem.at[s]).wait()
        pltpu.make_async_copy(eidx_hbm.at[0, pl.ds(0, CHUNK)], oidx.at[s],
                              pfsem.at[s]).wait()
        pltpu.make_async_copy(w_hbm.at[pl.ds(0, CHUNK)], wbuf.at[s],
                              pfsem.at[s]).wait()

    def start_gather(s, b):
        pltpu.async_copy(attr_hbm.at[0].at[aidx.at[s]], rows.at[b],
                         gsem.at[b])

    def wait_gather(b):
        pltpu.make_async_copy(attr_hbm.at[0, pl.ds(0, CHUNK)], rows.at[b],
                              gsem.at[b]).wait()

    def start_scatter(s, b):
        pltpu.async_copy(rows.at[b], acc_sh.at[oidx.at[s]], ssem.at[b],
                         add=True)

    def wait_scatter(s, b):
        pltpu.make_async_copy(rows.at[b], acc_sh.at[oidx.at[s]],
                              ssem.at[b]).wait()

    start_prefetch(0, 0)
    start_prefetch(1, 1)
    wait_prefetch(0)
    start_gather(0, 0)

    def chunk_body(g, carry):
        b = lax.rem(g, 2)
        nb = 1 - b
        s_cur = lax.rem(g, 3)
        s_nxt = lax.rem(g + 1, 3)
        s_pf = lax.rem(g + 2, 3)

        @pl.when(g + 1 < NFULL)
        def _():
            wait_prefetch(s_nxt)

            @pl.when(g >= 1)
            def _():
                wait_scatter(s_pf, nb)

            start_gather(s_nxt, nb)

        @pl.when(g + 2 < NFULL)
        def _():
            start_prefetch(g + 2, s_pf)

        wait_gather(b)
        _scale_rows(rows.at[b], wbuf.at[s_cur], CHUNK)
        start_scatter(s_cur, b)
        return carry

    lax.fori_loop(0, NFULL, chunk_body, 0)

    wait_scatter((NFULL - 2) % 3, (NFULL - 2) % 2)
    wait_scatter((NFULL - 1) % 3, (NFULL - 1) % 2)

    off = base + NFULL * CHUNK
    pltpu.sync_copy(eidx_hbm.at[1, pl.ds(off, REM)], aidx.at[0, pl.ds(0, REM)])
    pltpu.sync_copy(eidx_hbm.at[0, pl.ds(off, REM)], rem_oidx)
    pltpu.sync_copy(w_hbm.at[pl.ds(off, REM)], wbuf.at[0, pl.ds(0, REM)])
    pltpu.async_copy(attr_hbm.at[0].at[aidx.at[0, pl.ds(0, REM)]],
                     rows.at[0, pl.ds(0, REM)], rsem).wait()
    _scale_rows(rows.at[0], wbuf.at[0], REM)
    pltpu.sync_copy(rows.at[0, pl.ds(0, REM)], acc_sh.at[rem_oidx], add=True)

    plsc.subcore_barrier()
    pltpu.sync_copy(acc_sh.at[pl.ds(sid * rps, rps), pl.ds(0, D)],
                    out_f.at[cid, pl.ds(sid * rps, rps)])
    pltpu.sync_copy(acc_sh.at[pl.ds(sid * rps, rps), pl.ds(D, D_AUG - D)],
                    out_w.at[cid, pl.ds(sid * rps, rps)])


def _sc_aggregate(attr_aug, edge_index, edge_weight, zeros):
    mesh = plsc.VectorSubcoreMesh(core_axis_name="c", subcore_axis_name="s")
    return pl.kernel(
        _sc_body,
        out_type=(jax.ShapeDtypeStruct((NC, N_PAD, D), jnp.float32),
                  jax.ShapeDtypeStruct((NC, N_PAD, D_AUG - D), jnp.float32)),
        mesh=mesh,
        compiler_params=pltpu.CompilerParams(use_tc_tiling_on_sc=False),
        scratch_types=[
            pltpu.VMEM_SHARED((N_PAD, D_AUG), jnp.float32),
            pltpu.VMEM((3, CHUNK), jnp.int32),
            pltpu.VMEM((3, CHUNK), jnp.int32),
            pltpu.VMEM((3, CHUNK), jnp.float32),
            pltpu.VMEM((2, CHUNK, D_AUG), jnp.float32),
            pltpu.VMEM((REM,), jnp.int32),
            pltpu.SemaphoreType.DMA((2,)),
            pltpu.SemaphoreType.DMA((2,)),
            pltpu.SemaphoreType.DMA((3,)),
            pltpu.SemaphoreType.DMA,
        ],
    )(attr_aug, edge_index, edge_weight, zeros)


def _tc_body(accf_ref, accw_ref, obj_ref, wattr_t_ref, battr_ref,
             wproj_t_ref, bproj_ref, wupd_obj_t_ref, wupd_proj_t_ref,
             bupd_ref, out_ref):
    agg_raw = accf_ref[0] + accf_ref[1]
    wsum = accw_ref[0, :, :1] + accw_ref[1, :, :1]
    agg = agg_raw @ wattr_t_ref[...] + wsum * battr_ref[...]
    agg = agg / jnp.maximum(wsum, 1e-6)
    proj = agg @ wproj_t_ref[...] + bproj_ref[...]
    upd = obj_ref[0] @ wupd_obj_t_ref[...] + proj @ wupd_proj_t_ref[...]
    out_ref[0] = jnp.maximum(upd + bupd_ref[...], 0.0)


def _tc_epilogue(accf, accw, obj, W_attr, b_attr, W_proj, b_proj,
                 W_upd, b_upd):
    blk = 2000
    grid = (N_OBJ // blk,)
    return pl.pallas_call(
        _tc_body,
        grid=grid,
        in_specs=[
            pl.BlockSpec((NC, blk, D), lambda i: (0, i, 0)),
            pl.BlockSpec((NC, blk, D_AUG - D), lambda i: (0, i, 0)),
            pl.BlockSpec((1, blk, D), lambda i: (0, i, 0)),
            pl.BlockSpec((D, D), lambda i: (0, 0)),
            pl.BlockSpec((1, D), lambda i: (0, 0)),
            pl.BlockSpec((D, D), lambda i: (0, 0)),
            pl.BlockSpec((1, D), lambda i: (0, 0)),
            pl.BlockSpec((D, D), lambda i: (0, 0)),
            pl.BlockSpec((D, D), lambda i: (0, 0)),
            pl.BlockSpec((1, D), lambda i: (0, 0)),
        ],
        out_specs=pl.BlockSpec((1, blk, D), lambda i: (0, i, 0)),
        out_shape=jax.ShapeDtypeStruct((1, N_OBJ, D), jnp.float32),
    )(accf, accw, obj, W_attr.T, b_attr.reshape(1, D), W_proj.T,
      b_proj.reshape(1, D), W_upd[:, :D].T, W_upd[:, D:].T,
      b_upd.reshape(1, D))


@jax.jit
def kernel(object_feats, attr_feats, edge_index, edge_weight,
           W_attr, b_attr, W_proj, b_proj, W_upd, b_upd):
    aug = jnp.pad(attr_feats, ((0, 0), (0, 0), (0, D_AUG - D)),
                  constant_values=1.0)
    zeros = jnp.zeros((N_PAD, D_AUG), jnp.float32)

    accf, accw = _sc_aggregate(aug, edge_index, edge_weight, zeros)
    return _tc_epilogue(accf, accw, object_feats, W_attr, b_attr, W_proj,
                        b_proj, W_upd, b_upd)
